# baseline (speedup 1.0000x reference)
import jax
import jax.numpy as jnp
from jax import lax
from jax.experimental import pallas as pl
from jax.experimental.pallas import tpu as pltpu
from jax.experimental.pallas import tpu_sc as plsc

EMB_DIM = 64
NUM_NEG = 5
NC = 2
NS = 16
NW = NC * NS
LANES = 16
PER_TILE = 512
CHUNK = 64
GROUPS = CHUNK // LANES
NROWS = CHUNK * NUM_NEG
HALF_ROWS = 500000

_LN2 = 0.6931471805599453


def _softplus(x):
    t = jnp.exp(x)
    y = 1.0 + t
    b = lax.bitcast_convert_type(y, jnp.int32)
    e = (b >> 23) - 127
    m = lax.bitcast_convert_type((b & 0x007FFFFF) | 0x3F800000, jnp.float32)
    z = (m - 1.0) / (m + 1.0)
    z2 = z * z
    p = z * (2.0 + z2 * (0.66666667 + z2 * (0.4 + z2 * (0.28571429 + z2 * 0.22222222))))
    return e.astype(jnp.float32) * _LN2 + p


def _body(pos_u, pos_v, neg_f, u_emb, v2, out,
          pu_idx, pv_idx, ng_idx, pv_half, ng_half,
          u_buf0, u_buf1, v_buf0, v_buf1, n_buf0, n_buf1,
          acc_buf, sem0, sem1):
    wid = lax.axis_index("s") * NC + lax.axis_index("c")
    base = wid * PER_TILE

    pltpu.sync_copy(pos_u.at[pl.ds(base, PER_TILE)], pu_idx)
    pltpu.sync_copy(pos_v.at[pl.ds(base, PER_TILE)], pv_idx)
    pltpu.sync_copy(neg_f.at[pl.ds(base * NUM_NEG, PER_TILE * NUM_NEG)], ng_idx)

    def fold(i, _):
        r = pv_idx[pl.ds(i * LANES, LANES)]
        pv_half[pl.ds(i * LANES, LANES)] = r - jnp.where(
            r >= HALF_ROWS, HALF_ROWS, 0)
        return 0
    lax.fori_loop(0, PER_TILE // LANES, fold, 0)

    def fold_n(i, _):
        r = ng_idx[pl.ds(i * LANES, LANES)]
        ng_half[pl.ds(i * LANES, LANES)] = r - jnp.where(
            r >= HALF_ROWS, HALF_ROWS, 0)
        return 0
    lax.fori_loop(0, PER_TILE * NUM_NEG // LANES, fold_n, 0)

    u_bufs = (u_buf0, u_buf1)
    v_bufs = (v_buf0, v_buf1)
    n_bufs = (n_buf0, n_buf1)
    sems = (sem0, sem1)

    def fire(c, slot):
        ub, vb, nb = u_bufs[slot], v_bufs[slot], n_bufs[slot]
        sem = sems[slot]

        pltpu.async_copy(v2.at[pv_half.at[pl.ds(c * CHUNK, CHUNK)]], vb, sem)
        pltpu.async_copy(v2.at[ng_half.at[pl.ds(c * NROWS, 128)]],
                         nb.at[pl.ds(0, 128)], sem)
        pltpu.async_copy(v2.at[ng_half.at[pl.ds(c * NROWS + 128, 128)]],
                         nb.at[pl.ds(128, 128)], sem)
        pltpu.async_copy(v2.at[ng_half.at[pl.ds(c * NROWS + 256, 64)]],
                         nb.at[pl.ds(256, 64)], sem)

        def fire_u(g, _):
            uvec = pu_idx[pl.ds(c * CHUNK + g * LANES, LANES)]
            row0 = g * LANES
            for j in range(LANES):
                pltpu.async_copy(u_emb.at[uvec[j]], ub.at[row0 + j], sem)
            return 0

        lax.fori_loop(0, GROUPS, fire_u, 0)

    def drain(slot):
        pltpu.make_async_copy(u_emb.at[pl.ds(0, CHUNK)], u_bufs[slot], sems[slot]).wait()
        pltpu.make_async_copy(v2.at[pl.ds(0, CHUNK)], v_bufs[slot], sems[slot]).wait()
        pltpu.make_async_copy(v2.at[pl.ds(0, NROWS)], n_bufs[slot], sems[slot]).wait()

    lane_iota = lax.iota(jnp.int32, LANES)

    def compute(c, slot, acc):
        ub, vb, nb = u_bufs[slot], v_bufs[slot], n_bufs[slot]

        def group_step(g, acc):
            parv = pv_idx[pl.ds(c * CHUNK + g * LANES, LANES)] >= HALF_ROWS
            base_n = c * NROWS + g * LANES * NUM_NEG
            parn = [
                plsc.load_gather(
                    ng_idx, [base_n + lane_iota * NUM_NEG + n]) >= HALF_ROWS
                for n in range(NUM_NEG)
            ]

            def item_step(j, carry):
                sve, svo, ne, no = carry
                i = g * LANES + j
                u = [ub[i, pl.ds(k * LANES, LANES)] for k in range(4)]

                def dot2(ref, row):
                    pe = u[0] * ref[row, pl.ds(0, LANES)]
                    po = u[0] * ref[row, pl.ds(EMB_DIM, LANES)]
                    for k in range(1, 4):
                        pe = pe + u[k] * ref[row, pl.ds(k * LANES, LANES)]
                        po = po + u[k] * ref[row, pl.ds(EMB_DIM + k * LANES, LANES)]
                    return jnp.sum(pe), jnp.sum(po)

                msk = lane_iota == j
                se, so = dot2(vb, i)
                sve = jnp.where(msk, se, sve)
                svo = jnp.where(msk, so, svo)
                ne_out, no_out = [], []
                for n in range(NUM_NEG):
                    te, to = dot2(nb, i * NUM_NEG + n)
                    ne_out.append(jnp.where(msk, te, ne[n]))
                    no_out.append(jnp.where(msk, to, no[n]))
                return sve, svo, tuple(ne_out), tuple(no_out)

            z = jnp.zeros((LANES,), jnp.float32)
            z5 = (z, z, z, z, z)
            sve, svo, ne, no = lax.fori_loop(
                0, LANES, item_step, (z, z, z5, z5))

            s = jnp.where(parv, svo, sve)
            acc = acc + _softplus(-jnp.clip(s, -10.0, 10.0))
            for n in range(NUM_NEG):
                t = jnp.where(parn[n], no[n], ne[n])
                acc = acc + _softplus(jnp.clip(t, -10.0, 10.0))
            return acc

        return lax.fori_loop(0, GROUPS, group_step, acc)

    n_chunks = PER_TILE // CHUNK
    acc = jnp.zeros((LANES,), jnp.float32)
    fire(0, 0)
    for c in range(n_chunks):
        if c + 1 < n_chunks:
            fire(c + 1, (c + 1) % 2)
        drain(c % 2)
        acc = compute(c, c % 2, acc)

    acc_buf[...] = acc
    pltpu.sync_copy(acc_buf, out.at[wid])


@jax.jit
def _sc_skipgram(pos_u, pos_v, neg_f, u_emb, v2):
    mesh = plsc.VectorSubcoreMesh(core_axis_name="c", subcore_axis_name="s")
    kcall = pl.kernel(
        _body,
        out_type=jax.ShapeDtypeStruct((NW, LANES), jnp.float32),
        mesh=mesh,
        compiler_params=pltpu.CompilerParams(needs_layout_passes=False),
        scratch_types=[
            pltpu.VMEM((PER_TILE,), jnp.int32),
            pltpu.VMEM((PER_TILE,), jnp.int32),
            pltpu.VMEM((PER_TILE * NUM_NEG,), jnp.int32),
            pltpu.VMEM((PER_TILE,), jnp.int32),
            pltpu.VMEM((PER_TILE * NUM_NEG,), jnp.int32),
            pltpu.VMEM((CHUNK, EMB_DIM), jnp.float32),
            pltpu.VMEM((CHUNK, EMB_DIM), jnp.float32),
            pltpu.VMEM((CHUNK, 2 * EMB_DIM), jnp.float32),
            pltpu.VMEM((CHUNK, 2 * EMB_DIM), jnp.float32),
            pltpu.VMEM((NROWS, 2 * EMB_DIM), jnp.float32),
            pltpu.VMEM((NROWS, 2 * EMB_DIM), jnp.float32),
            pltpu.VMEM((LANES,), jnp.float32),
            pltpu.SemaphoreType.DMA,
            pltpu.SemaphoreType.DMA,
        ],
    )
    return kcall(pos_u, pos_v, neg_f, u_emb, v2)


_REPACK_ROWS = 1000


def _repack_body(a_ref, b_ref, o_ref):
    o_ref[:, 0:EMB_DIM] = a_ref[...]
    o_ref[:, EMB_DIM:2 * EMB_DIM] = b_ref[...]


def _repack(v_emb):
    half = v_emb.shape[0] // 2
    grid = half // _REPACK_ROWS
    return pl.pallas_call(
        _repack_body,
        grid=(grid,),
        in_specs=[
            pl.BlockSpec((_REPACK_ROWS, EMB_DIM), lambda i: (i, 0)),
            pl.BlockSpec((_REPACK_ROWS, EMB_DIM), lambda i: (i + grid, 0)),
        ],
        out_specs=pl.BlockSpec((_REPACK_ROWS, 2 * EMB_DIM), lambda i: (i, 0)),
        out_shape=jax.ShapeDtypeStruct((half, 2 * EMB_DIM), jnp.float32),
    )(v_emb, v_emb)


def kernel(pos_u, pos_v, neg_v, u_emb, v_emb):
    batch = pos_u.shape[0]
    neg_f = neg_v.astype(jnp.int32).reshape(-1)
    v2 = _repack(v_emb)
    partials = _sc_skipgram(pos_u.astype(jnp.int32), pos_v.astype(jnp.int32),
                            neg_f, u_emb, v2)
    return jnp.sum(partials) * (1.0 / batch)


# speedup vs baseline: 1.2001x; 1.2001x over previous
import jax
import jax.numpy as jnp
from jax import lax
from jax.experimental import pallas as pl
from jax.experimental.pallas import tpu as pltpu
from jax.experimental.pallas import tpu_sc as plsc

EMB_DIM = 64
NUM_NEG = 5
NC = 2
NS = 16
NW = NC * NS
LANES = 16
PER_TILE = 512
CHUNK = 64
GROUPS = CHUNK // LANES
NROWS = CHUNK * NUM_NEG
HALF_ROWS = 500000

_LN2 = 0.6931471805599453


def _softplus(x):
    t = jnp.exp(x)
    y = 1.0 + t
    b = lax.bitcast_convert_type(y, jnp.int32)
    e = (b >> 23) - 127
    m = lax.bitcast_convert_type((b & 0x007FFFFF) | 0x3F800000, jnp.float32)
    z = (m - 1.0) / (m + 1.0)
    z2 = z * z
    p = z * (2.0 + z2 * (0.66666667 + z2 * (0.4 + z2 * (0.28571429 + z2 * 0.22222222))))
    return e.astype(jnp.float32) * _LN2 + p


def _body(pos_u, pos_v, neg_f, u_emb, v2, out,
          pu_idx, pv_idx, ng_idx, pv_half, ng_half,
          u_buf0, u_buf1, v_buf0, v_buf1, n_buf0, n_buf1,
          acc_buf, sem0, sem1):
    wid = lax.axis_index("s") * NC + lax.axis_index("c")
    base = wid * PER_TILE

    pltpu.sync_copy(pos_u.at[pl.ds(base, PER_TILE)], pu_idx)
    pltpu.sync_copy(pos_v.at[pl.ds(base, PER_TILE)], pv_idx)
    pltpu.sync_copy(neg_f.at[pl.ds(base * NUM_NEG, PER_TILE * NUM_NEG)], ng_idx)

    def fold(i, _):
        r = pv_idx[pl.ds(i * LANES, LANES)]
        pv_half[pl.ds(i * LANES, LANES)] = r - jnp.where(
            r >= HALF_ROWS, HALF_ROWS, 0)
        return 0
    lax.fori_loop(0, PER_TILE // LANES, fold, 0)

    def fold_n(i, _):
        r = ng_idx[pl.ds(i * LANES, LANES)]
        ng_half[pl.ds(i * LANES, LANES)] = r - jnp.where(
            r >= HALF_ROWS, HALF_ROWS, 0)
        return 0
    lax.fori_loop(0, PER_TILE * NUM_NEG // LANES, fold_n, 0)

    u_bufs = (u_buf0, u_buf1)
    v_bufs = (v_buf0, v_buf1)
    n_bufs = (n_buf0, n_buf1)
    sems = (sem0, sem1)

    def fire(c, slot):
        ub, vb, nb = u_bufs[slot], v_bufs[slot], n_bufs[slot]
        sem = sems[slot]

        pltpu.async_copy(v2.at[pv_half.at[pl.ds(c * CHUNK, CHUNK)]], vb, sem)
        pltpu.async_copy(v2.at[ng_half.at[pl.ds(c * NROWS, 128)]],
                         nb.at[pl.ds(0, 128)], sem)
        pltpu.async_copy(v2.at[ng_half.at[pl.ds(c * NROWS + 128, 128)]],
                         nb.at[pl.ds(128, 128)], sem)
        pltpu.async_copy(v2.at[ng_half.at[pl.ds(c * NROWS + 256, 64)]],
                         nb.at[pl.ds(256, 64)], sem)

        def fire_u(g, _):
            uvec = pu_idx[pl.ds(c * CHUNK + g * LANES, LANES)]
            row0 = g * LANES
            for j in range(LANES):
                pltpu.async_copy(u_emb.at[uvec[j]], ub.at[row0 + j], sem)
            return 0

        lax.fori_loop(0, GROUPS, fire_u, 0)

    def drain(slot):
        pltpu.make_async_copy(u_emb.at[pl.ds(0, CHUNK)], u_bufs[slot], sems[slot]).wait()
        pltpu.make_async_copy(v2.at[pl.ds(0, CHUNK)], v_bufs[slot], sems[slot]).wait()
        pltpu.make_async_copy(v2.at[pl.ds(0, NROWS)], n_bufs[slot], sems[slot]).wait()

    lane_iota = lax.iota(jnp.int32, LANES)

    def compute(c, slot, acc):
        ub, vb, nb = u_bufs[slot], v_bufs[slot], n_bufs[slot]

        def group_step(g, acc):
            parv = pv_idx[pl.ds(c * CHUNK + g * LANES, LANES)] >= HALF_ROWS
            base_n = c * NROWS + g * LANES * NUM_NEG
            parn = [
                plsc.load_gather(
                    ng_idx, [base_n + lane_iota * NUM_NEG + n]) >= HALF_ROWS
                for n in range(NUM_NEG)
            ]

            def item_step(j, carry):
                sve, svo, ne, no = carry
                i = g * LANES + j
                u = [ub[i, pl.ds(k * LANES, LANES)] for k in range(4)]

                def dot2(ref, row):
                    pe = u[0] * ref[row, pl.ds(0, LANES)]
                    po = u[0] * ref[row, pl.ds(EMB_DIM, LANES)]
                    for k in range(1, 4):
                        pe = pe + u[k] * ref[row, pl.ds(k * LANES, LANES)]
                        po = po + u[k] * ref[row, pl.ds(EMB_DIM + k * LANES, LANES)]
                    return jnp.sum(pe), jnp.sum(po)

                msk = lane_iota == j
                se, so = dot2(vb, i)
                sve = jnp.where(msk, se, sve)
                svo = jnp.where(msk, so, svo)
                ne_out, no_out = [], []
                for n in range(NUM_NEG):
                    te, to = dot2(nb, i * NUM_NEG + n)
                    ne_out.append(jnp.where(msk, te, ne[n]))
                    no_out.append(jnp.where(msk, to, no[n]))
                return sve, svo, tuple(ne_out), tuple(no_out)

            z = jnp.zeros((LANES,), jnp.float32)
            z5 = (z, z, z, z, z)
            sve, svo, ne, no = lax.fori_loop(
                0, LANES, item_step, (z, z, z5, z5))

            s = jnp.where(parv, svo, sve)
            acc = acc + _softplus(-jnp.clip(s, -10.0, 10.0))
            for n in range(NUM_NEG):
                t = jnp.where(parn[n], no[n], ne[n])
                acc = acc + _softplus(jnp.clip(t, -10.0, 10.0))
            return acc

        return lax.fori_loop(0, GROUPS, group_step, acc)

    n_chunks = PER_TILE // CHUNK
    acc = jnp.zeros((LANES,), jnp.float32)
    fire(0, 0)
    for c in range(n_chunks):
        if c + 1 < n_chunks:
            fire(c + 1, (c + 1) % 2)
        drain(c % 2)
        acc = compute(c, c % 2, acc)

    acc_buf[...] = acc
    pltpu.sync_copy(acc_buf, out.at[wid])


@jax.jit
def _sc_skipgram(pos_u, pos_v, neg_f, u_emb, v2):
    mesh = plsc.VectorSubcoreMesh(core_axis_name="c", subcore_axis_name="s")
    kcall = pl.kernel(
        _body,
        out_type=jax.ShapeDtypeStruct((NW, LANES), jnp.float32),
        mesh=mesh,
        compiler_params=pltpu.CompilerParams(needs_layout_passes=False),
        scratch_types=[
            pltpu.VMEM((PER_TILE,), jnp.int32),
            pltpu.VMEM((PER_TILE,), jnp.int32),
            pltpu.VMEM((PER_TILE * NUM_NEG,), jnp.int32),
            pltpu.VMEM((PER_TILE,), jnp.int32),
            pltpu.VMEM((PER_TILE * NUM_NEG,), jnp.int32),
            pltpu.VMEM((CHUNK, EMB_DIM), jnp.float32),
            pltpu.VMEM((CHUNK, EMB_DIM), jnp.float32),
            pltpu.VMEM((CHUNK, 2 * EMB_DIM), jnp.float32),
            pltpu.VMEM((CHUNK, 2 * EMB_DIM), jnp.float32),
            pltpu.VMEM((NROWS, 2 * EMB_DIM), jnp.float32),
            pltpu.VMEM((NROWS, 2 * EMB_DIM), jnp.float32),
            pltpu.VMEM((LANES,), jnp.float32),
            pltpu.SemaphoreType.DMA,
            pltpu.SemaphoreType.DMA,
        ],
    )
    return kcall(pos_u, pos_v, neg_f, u_emb, v2)


_REPACK_ROWS = 5000


def _repack_body(a_ref, b_ref, o_ref):
    o_ref[:, 0:EMB_DIM] = a_ref[...]
    o_ref[:, EMB_DIM:2 * EMB_DIM] = b_ref[...]


def _repack(v_emb):
    half = v_emb.shape[0] // 2
    grid = half // _REPACK_ROWS
    return pl.pallas_call(
        _repack_body,
        grid=(grid,),
        in_specs=[
            pl.BlockSpec((_REPACK_ROWS, EMB_DIM), lambda i: (i, 0)),
            pl.BlockSpec((_REPACK_ROWS, EMB_DIM), lambda i: (i + grid, 0)),
        ],
        out_specs=pl.BlockSpec((_REPACK_ROWS, 2 * EMB_DIM), lambda i: (i, 0)),
        out_shape=jax.ShapeDtypeStruct((half, 2 * EMB_DIM), jnp.float32),
    )(v_emb, v_emb)


def kernel(pos_u, pos_v, neg_v, u_emb, v_emb):
    batch = pos_u.shape[0]
    neg_f = neg_v.astype(jnp.int32).reshape(-1)
    v2 = _repack(v_emb)
    partials = _sc_skipgram(pos_u.astype(jnp.int32), pos_v.astype(jnp.int32),
                            neg_f, u_emb, v2)
    return jnp.sum(partials) * (1.0 / batch)


# speedup vs baseline: 1.6051x; 1.3374x over previous
import jax
import jax.numpy as jnp
from jax import lax
from jax.experimental import pallas as pl
from jax.experimental.pallas import tpu as pltpu
from jax.experimental.pallas import tpu_sc as plsc

EMB_DIM = 64
NUM_NEG = 5
NC = 2
NS = 16
NW = NC * NS
LANES = 16
PER_TILE = 512
CHUNK = 64
GROUPS = CHUNK // LANES
NROWS = CHUNK * NUM_NEG

_LN2 = 0.6931471805599453


def _softplus(x):
    t = jnp.exp(x)
    y = 1.0 + t
    b = lax.bitcast_convert_type(y, jnp.int32)
    e = (b >> 23) - 127
    m = lax.bitcast_convert_type((b & 0x007FFFFF) | 0x3F800000, jnp.float32)
    z = (m - 1.0) / (m + 1.0)
    z2 = z * z
    p = z * (2.0 + z2 * (0.66666667 + z2 * (0.4 + z2 * (0.28571429 + z2 * 0.22222222))))
    return e.astype(jnp.float32) * _LN2 + p


def _body(pos_u, pos_v, neg_f, u_emb, v_emb, out,
          pu_idx, pv_idx, ng_idx,
          u_buf0, u_buf1, v_buf0, v_buf1, n_buf0, n_buf1,
          acc_buf,
          semu0, semu1, semv0, semv1, semn0, semn1):
    wid = lax.axis_index("s") * NC + lax.axis_index("c")
    base = wid * PER_TILE

    pltpu.sync_copy(pos_u.at[pl.ds(base, PER_TILE)], pu_idx)
    pltpu.sync_copy(pos_v.at[pl.ds(base, PER_TILE)], pv_idx)
    pltpu.sync_copy(neg_f.at[pl.ds(base * NUM_NEG, PER_TILE * NUM_NEG)], ng_idx)

    u_bufs = (u_buf0, u_buf1)
    v_bufs = (v_buf0, v_buf1)
    n_bufs = (n_buf0, n_buf1)
    semus = (semu0, semu1)
    semvs = (semv0, semv1)
    semns = (semn0, semn1)

    def fire(c, slot):
        ub, vb, nb = u_bufs[slot], v_bufs[slot], n_bufs[slot]

        def fire_uv(g, _):
            uvec = pu_idx[pl.ds(c * CHUNK + g * LANES, LANES)]
            vvec = pv_idx[pl.ds(c * CHUNK + g * LANES, LANES)]
            row0 = g * LANES
            for j in range(LANES):
                pltpu.async_copy(u_emb.at[uvec[j]], ub.at[row0 + j], semus[slot])
                pltpu.async_copy(v_emb.at[vvec[j]], vb.at[row0 + j], semvs[slot])
            return 0

        lax.fori_loop(0, GROUPS, fire_uv, 0)

        def fire_n(k, _):
            nvec = ng_idx[pl.ds(c * NROWS + k * LANES, LANES)]
            row0 = k * LANES
            for j in range(LANES):
                pltpu.async_copy(v_emb.at[nvec[j]], nb.at[row0 + j], semns[slot])
            return 0

        lax.fori_loop(0, NROWS // LANES, fire_n, 0)

    def drain(slot):
        pltpu.make_async_copy(u_emb.at[pl.ds(0, CHUNK)], u_bufs[slot], semus[slot]).wait()
        pltpu.make_async_copy(u_emb.at[pl.ds(0, CHUNK)], v_bufs[slot], semvs[slot]).wait()
        pltpu.make_async_copy(u_emb.at[pl.ds(0, NROWS)], n_bufs[slot], semns[slot]).wait()

    lane_iota = lax.iota(jnp.int32, LANES)

    def compute(slot, acc):
        ub, vb, nb = u_bufs[slot], v_bufs[slot], n_bufs[slot]

        def group_step(g, acc):
            def item_step(j, carry):
                sv, n0, n1, n2, n3, n4 = carry
                i = g * LANES + j
                u = [ub[i, pl.ds(k * LANES, LANES)] for k in range(4)]

                def dot(ref, row):
                    p = u[0] * ref[row, pl.ds(0, LANES)]
                    for k in range(1, 4):
                        p = p + u[k] * ref[row, pl.ds(k * LANES, LANES)]
                    return jnp.sum(p)

                msk = lane_iota == j
                sv = jnp.where(msk, dot(vb, i), sv)
                outs = []
                for n, cur in enumerate((n0, n1, n2, n3, n4)):
                    outs.append(jnp.where(msk, dot(nb, i * NUM_NEG + n), cur))
                return (sv, *outs)

            z = jnp.zeros((LANES,), jnp.float32)
            sv, n0, n1, n2, n3, n4 = lax.fori_loop(
                0, LANES, item_step, (z, z, z, z, z, z))
            acc = acc + _softplus(-jnp.clip(sv, -10.0, 10.0))
            for nk in (n0, n1, n2, n3, n4):
                acc = acc + _softplus(jnp.clip(nk, -10.0, 10.0))
            return acc

        return lax.fori_loop(0, GROUPS, group_step, acc)

    n_chunks = PER_TILE // CHUNK
    acc = jnp.zeros((LANES,), jnp.float32)
    fire(0, 0)
    for c in range(n_chunks):
        if c + 1 < n_chunks:
            fire(c + 1, (c + 1) % 2)
        drain(c % 2)
        acc = compute(c % 2, acc)

    acc_buf[...] = acc
    pltpu.sync_copy(acc_buf, out.at[wid])


@jax.jit
def _sc_skipgram(pos_u, pos_v, neg_f, u_emb, v_emb):
    mesh = plsc.VectorSubcoreMesh(core_axis_name="c", subcore_axis_name="s")
    kcall = pl.kernel(
        _body,
        out_type=jax.ShapeDtypeStruct((NW, LANES), jnp.float32),
        mesh=mesh,
        compiler_params=pltpu.CompilerParams(needs_layout_passes=False),
        scratch_types=[
            pltpu.VMEM((PER_TILE,), jnp.int32),
            pltpu.VMEM((PER_TILE,), jnp.int32),
            pltpu.VMEM((PER_TILE * NUM_NEG,), jnp.int32),
            pltpu.VMEM((CHUNK, EMB_DIM), jnp.float32),
            pltpu.VMEM((CHUNK, EMB_DIM), jnp.float32),
            pltpu.VMEM((CHUNK, EMB_DIM), jnp.float32),
            pltpu.VMEM((CHUNK, EMB_DIM), jnp.float32),
            pltpu.VMEM((NROWS, EMB_DIM), jnp.float32),
            pltpu.VMEM((NROWS, EMB_DIM), jnp.float32),
            pltpu.VMEM((LANES,), jnp.float32),
            pltpu.SemaphoreType.DMA,
            pltpu.SemaphoreType.DMA,
            pltpu.SemaphoreType.DMA,
            pltpu.SemaphoreType.DMA,
            pltpu.SemaphoreType.DMA,
            pltpu.SemaphoreType.DMA,
        ],
    )
    return kcall(pos_u, pos_v, neg_f, u_emb, v_emb)


def kernel(pos_u, pos_v, neg_v, u_emb, v_emb):
    batch = pos_u.shape[0]
    neg_f = neg_v.astype(jnp.int32).reshape(-1)
    partials = _sc_skipgram(pos_u.astype(jnp.int32), pos_v.astype(jnp.int32),
                            neg_f, u_emb, v_emb)
    return jnp.sum(partials) * (1.0 / batch)
